# baseline (device time: 27705 ns/iter reference)
import os

import jax
import jax.numpy as jnp
from jax import lax
from jax.experimental import pallas as pl
from jax.experimental.pallas import tpu as pltpu

_PHASES = int(os.environ.get("KERNEL_PHASES", "2"))

N_DEV = 16
N_TOK = 512
D_IN = 256
D_OUT = 512
E_LOCAL = 4
N_EXP = 64
P = 4
QROWS = N_TOK // P
ROWS = QROWS // P


def kernel(x, router_W, route_idx, expert_W, shared_W):
    def body(
        x_ref,
        rw_ref,
        idx_ref,
        ew_ref,
        sw_ref,
        out_ref,
        acc_ref,
        rs1_buf,
        q_ref,
        rs2_buf,
        ssem,
        rsem,
    ):
        my = lax.axis_index("i")
        z = my // P
        k = my % P

        def plane_peer(off):
            return P * z + (k + off) % P

        def z_peer(off):
            return P * ((z + off) % P) + k

        if _PHASES > 0:
            barrier_sem = pltpu.get_barrier_semaphore()
            for off in range(1, P):
                for peer in (plane_peer(off), z_peer(off)):
                    pl.semaphore_signal(
                        barrier_sem,
                        inc=1,
                        device_id=(peer,),
                        device_id_type=pl.DeviceIdType.MESH,
                    )

        xb = x_ref[...].astype(jnp.bfloat16)
        scores = jnp.dot(
            xb, rw_ref[...].astype(jnp.bfloat16), preferred_element_type=jnp.float32
        )
        s_max = jnp.max(scores, axis=-1, keepdims=True)
        e_s = jnp.exp(scores - s_max)
        probs = e_s / jnp.sum(e_s, axis=-1, keepdims=True)
        idx = idx_ref[...]
        cols = lax.broadcasted_iota(jnp.int32, (N_TOK, N_EXP), 1)
        p_chosen = jnp.sum(
            jnp.where(cols == idx, probs, 0.0), axis=-1, keepdims=True
        )

        xw = jnp.concatenate(
            [
                xb
                * jnp.where(idx == my * E_LOCAL + e, p_chosen, 0.0).astype(
                    jnp.bfloat16
                )
                for e in range(E_LOCAL)
            ],
            axis=1,
        )
        wm = ew_ref[...].astype(jnp.bfloat16).reshape(E_LOCAL * D_IN, D_OUT)
        partial = jnp.dot(xw, wm, preferred_element_type=jnp.float32)
        acc_ref[...] = partial.reshape(P, QROWS, D_OUT).astype(jnp.bfloat16)

        if _PHASES == 0:
            out_ref[...] = acc_ref[...].reshape(N_TOK, D_OUT)
            return

        pl.semaphore_wait(barrier_sem, 2 * (P - 1))

        def l1_copy(off):
            return pltpu.make_async_remote_copy(
                src_ref=acc_ref.at[(k + off) % P],
                dst_ref=rs1_buf.at[off - 1],
                send_sem=ssem.at[off - 1],
                recv_sem=rsem.at[off - 1],
                device_id=(plane_peer(off),),
                device_id_type=pl.DeviceIdType.MESH,
            )

        l1 = [l1_copy(off) for off in range(1, P)]
        for c in l1:
            c.start()

        c_rows = QROWS * k + ROWS * z
        shared_own = jnp.dot(
            x_ref[pl.ds(c_rows, ROWS), :].astype(jnp.bfloat16),
            sw_ref[...].astype(jnp.bfloat16),
            preferred_element_type=jnp.float32,
        )

        for c in l1:
            c.wait_send()
        for c in l1:
            c.wait_recv()

        quarter = acc_ref[k].astype(jnp.float32) + jnp.sum(
            rs1_buf[...].astype(jnp.float32), axis=0
        )
        q_ref[...] = quarter.reshape(P, ROWS, D_OUT).astype(jnp.bfloat16)

        def l2_copy(off):
            return pltpu.make_async_remote_copy(
                src_ref=q_ref.at[(z + off) % P],
                dst_ref=rs2_buf.at[off - 1],
                send_sem=ssem.at[3 + off - 1],
                recv_sem=rsem.at[3 + off - 1],
                device_id=(z_peer(off),),
                device_id_type=pl.DeviceIdType.MESH,
            )

        l2 = [l2_copy(off) for off in range(1, P)]
        for c in l2:
            c.start()
        for c in l2:
            c.wait_send()
        for c in l2:
            c.wait_recv()

        red = (
            q_ref[z].astype(jnp.float32)
            + jnp.sum(rs2_buf[...].astype(jnp.float32), axis=0)
            + shared_own
        )
        out_ref[pl.ds(c_rows, ROWS), :] = red.astype(jnp.bfloat16)

        if _PHASES == 1:
            return

        g1 = []
        for off in range(1, P):
            c = pltpu.make_async_remote_copy(
                src_ref=out_ref.at[pl.ds(c_rows, ROWS)],
                dst_ref=out_ref.at[pl.ds(c_rows, ROWS)],
                send_sem=ssem.at[6 + off - 1],
                recv_sem=rsem.at[6 + off - 1],
                device_id=(z_peer(off),),
                device_id_type=pl.DeviceIdType.MESH,
            )
            c.start()
            g1.append(c)
        for c in g1:
            c.wait_send()
        for c in g1:
            c.wait_recv()

        g2 = []
        for off in range(1, P):
            c = pltpu.make_async_remote_copy(
                src_ref=out_ref.at[pl.ds(QROWS * k, QROWS)],
                dst_ref=out_ref.at[pl.ds(QROWS * k, QROWS)],
                send_sem=ssem.at[9 + off - 1],
                recv_sem=rsem.at[9 + off - 1],
                device_id=(plane_peer(off),),
                device_id_type=pl.DeviceIdType.MESH,
            )
            c.start()
            g2.append(c)
        for c in g2:
            c.wait_send()
        for c in g2:
            c.wait_recv()

    return pl.pallas_call(
        body,
        out_shape=jax.ShapeDtypeStruct((N_TOK, D_OUT), jnp.bfloat16),
        in_specs=[pl.BlockSpec(memory_space=pltpu.VMEM)] * 5,
        out_specs=pl.BlockSpec(memory_space=pltpu.VMEM),
        scratch_shapes=[
            pltpu.VMEM((P, QROWS, D_OUT), jnp.bfloat16),
            pltpu.VMEM((P - 1, QROWS, D_OUT), jnp.bfloat16),
            pltpu.VMEM((P, ROWS, D_OUT), jnp.bfloat16),
            pltpu.VMEM((P - 1, ROWS, D_OUT), jnp.bfloat16),
            pltpu.SemaphoreType.DMA((12,)),
            pltpu.SemaphoreType.DMA((12,)),
        ],
        **(
            {"compiler_params": pltpu.CompilerParams(collective_id=0)}
            if _PHASES > 0
            else {}
        ),
    )(x, router_W, route_idx, expert_W, shared_W)


# device time: 26198 ns/iter; 1.0575x vs baseline; 1.0575x over previous
import os

import jax
import jax.numpy as jnp
from jax import lax
from jax.experimental import pallas as pl
from jax.experimental.pallas import tpu as pltpu

_PHASES = int(os.environ.get("KERNEL_PHASES", "2"))

N_DEV = 16
N_TOK = 512
D_IN = 256
D_OUT = 512
E_LOCAL = 4
N_EXP = 64
ROWS = N_TOK // N_DEV
N_HALF = 2
COLS = D_OUT // N_HALF
N_PEER = N_DEV - 1


def kernel(x, router_W, route_idx, expert_W, shared_W):
    def body(
        x_ref,
        rw_ref,
        idx_ref,
        ew_ref,
        sw_ref,
        out_ref,
        acc_ref,
        rs_buf,
        rs_ssem,
        rs_rsem,
        ag_ssem,
        ag_rsem,
    ):
        my = lax.axis_index("i")

        if _PHASES > 0:
            barrier_sem = pltpu.get_barrier_semaphore()
            for d in range(1, N_DEV):
                pl.semaphore_signal(
                    barrier_sem,
                    inc=1,
                    device_id=((my + d) % N_DEV,),
                    device_id_type=pl.DeviceIdType.MESH,
                )

        xb = x_ref[...].astype(jnp.bfloat16)
        scores = jnp.dot(
            xb, rw_ref[...].astype(jnp.bfloat16), preferred_element_type=jnp.float32
        )
        s_max = jnp.max(scores, axis=-1, keepdims=True)
        e_s = jnp.exp(scores - s_max)
        probs = e_s / jnp.sum(e_s, axis=-1, keepdims=True)
        idx = idx_ref[...]
        cols = lax.broadcasted_iota(jnp.int32, (N_TOK, N_EXP), 1)
        p_chosen = jnp.sum(
            jnp.where(cols == idx, probs, 0.0), axis=-1, keepdims=True
        )

        xw = jnp.concatenate(
            [
                xb
                * jnp.where(idx == my * E_LOCAL + e, p_chosen, 0.0).astype(
                    jnp.bfloat16
                )
                for e in range(E_LOCAL)
            ],
            axis=1,
        )
        wm = ew_ref[...].astype(jnp.bfloat16).reshape(E_LOCAL * D_IN, D_OUT)

        def rs_copy(h, d):
            return pltpu.make_async_remote_copy(
                src_ref=acc_ref.at[h * N_DEV + (my + d) % N_DEV],
                dst_ref=rs_buf.at[h * N_PEER + d - 1],
                send_sem=rs_ssem.at[h * N_PEER + d - 1],
                recv_sem=rs_rsem.at[h * N_PEER + d - 1],
                device_id=((my + d) % N_DEV,),
                device_id_type=pl.DeviceIdType.MESH,
            )

        for h in range(N_HALF):
            partial_h = jnp.dot(
                xw, wm[:, h * COLS : (h + 1) * COLS],
                preferred_element_type=jnp.float32,
            )
            acc_ref[h * N_DEV : (h + 1) * N_DEV] = partial_h.reshape(
                N_DEV, ROWS, COLS
            ).astype(jnp.bfloat16)

            if _PHASES == 0:
                continue
            if h == 0:
                pl.semaphore_wait(barrier_sem, N_DEV - 1)
            for d in range(1, N_DEV):
                rs_copy(h, d).start()

        if _PHASES == 0:
            out_ref[...] = jnp.concatenate(
                [
                    acc_ref[h * N_DEV : (h + 1) * N_DEV].reshape(N_TOK, COLS)
                    for h in range(N_HALF)
                ],
                axis=1,
            )
            return

        shared_own = jnp.dot(
            x_ref[pl.ds(my * ROWS, ROWS), :].astype(jnp.bfloat16),
            sw_ref[...].astype(jnp.bfloat16),
            preferred_element_type=jnp.float32,
        )

        ag = []
        for h in range(N_HALF):
            rs = [rs_copy(h, d) for d in range(1, N_DEV)]
            for c in rs:
                c.wait_send()
            for c in rs:
                c.wait_recv()

            own = acc_ref[my + h * N_DEV].astype(jnp.float32)
            red = (
                own
                + jnp.sum(
                    rs_buf[h * N_PEER : (h + 1) * N_PEER].astype(jnp.float32),
                    axis=0,
                )
                + shared_own[:, h * COLS : (h + 1) * COLS]
            )
            out_ref[pl.ds(my * ROWS, ROWS), h * COLS : (h + 1) * COLS] = red.astype(
                jnp.bfloat16
            )

            if _PHASES == 1:
                continue
            for d in range(1, N_DEV):
                c = pltpu.make_async_remote_copy(
                    src_ref=out_ref.at[
                        pl.ds(my * ROWS, ROWS), h * COLS : (h + 1) * COLS
                    ],
                    dst_ref=out_ref.at[
                        pl.ds(my * ROWS, ROWS), h * COLS : (h + 1) * COLS
                    ],
                    send_sem=ag_ssem.at[h * N_PEER + d - 1],
                    recv_sem=ag_rsem.at[h * N_PEER + d - 1],
                    device_id=((my + d) % N_DEV,),
                    device_id_type=pl.DeviceIdType.MESH,
                )
                c.start()
                ag.append(c)

        for c in ag:
            c.wait_send()
        for c in ag:
            c.wait_recv()

    return pl.pallas_call(
        body,
        out_shape=jax.ShapeDtypeStruct((N_TOK, D_OUT), jnp.bfloat16),
        in_specs=[pl.BlockSpec(memory_space=pltpu.VMEM)] * 5,
        out_specs=pl.BlockSpec(memory_space=pltpu.VMEM),
        scratch_shapes=[
            pltpu.VMEM((N_HALF * N_DEV, ROWS, COLS), jnp.bfloat16),
            pltpu.VMEM((N_HALF * N_PEER, ROWS, COLS), jnp.bfloat16),
            pltpu.SemaphoreType.DMA((N_HALF * N_PEER,)),
            pltpu.SemaphoreType.DMA((N_HALF * N_PEER,)),
            pltpu.SemaphoreType.DMA((N_HALF * N_PEER,)),
            pltpu.SemaphoreType.DMA((N_HALF * N_PEER,)),
        ],
        **(
            {"compiler_params": pltpu.CompilerParams(collective_id=0)}
            if _PHASES > 0
            else {}
        ),
    )(x, router_W, route_idx, expert_W, shared_W)


# device time: 25985 ns/iter; 1.0662x vs baseline; 1.0082x over previous
import os

import jax
import jax.numpy as jnp
from jax import lax
from jax.experimental import pallas as pl
from jax.experimental.pallas import tpu as pltpu

_PHASES = int(os.environ.get("KERNEL_PHASES", "2"))

N_DEV = 16
N_TOK = 512
D_IN = 256
D_OUT = 512
E_LOCAL = 4
N_EXP = 64
ROWS = N_TOK // N_DEV
N_HALF = 2
COLS = D_OUT // N_HALF
N_PEER = N_DEV - 1


def kernel(x, router_W, route_idx, expert_W, shared_W):
    def body(
        x_ref,
        rw_ref,
        idx_ref,
        ew_ref,
        sw_ref,
        out_ref,
        acc_ref,
        rs_buf,
        ew_vmem,
        sw_vmem,
        in_sems,
        rs_ssem,
        rs_rsem,
        ag_ssem,
        ag_rsem,
    ):
        my = lax.axis_index("i")

        ew_dma = pltpu.make_async_copy(ew_ref, ew_vmem, in_sems.at[0])
        ew_dma.start()
        sw_dma = pltpu.make_async_copy(sw_ref, sw_vmem, in_sems.at[1])
        sw_dma.start()

        if _PHASES > 0:
            barrier_sem = pltpu.get_barrier_semaphore()
            for d in range(1, N_DEV):
                pl.semaphore_signal(
                    barrier_sem,
                    inc=1,
                    device_id=((my + d) % N_DEV,),
                    device_id_type=pl.DeviceIdType.MESH,
                )

        xb = x_ref[...].astype(jnp.bfloat16)
        scores = jnp.dot(
            xb, rw_ref[...].astype(jnp.bfloat16), preferred_element_type=jnp.float32
        )
        s_max = jnp.max(scores, axis=-1, keepdims=True)
        e_s = jnp.exp(scores - s_max)
        probs = e_s / jnp.sum(e_s, axis=-1, keepdims=True)
        idx = idx_ref[...]
        cols = lax.broadcasted_iota(jnp.int32, (N_TOK, N_EXP), 1)
        p_chosen = jnp.sum(
            jnp.where(cols == idx, probs, 0.0), axis=-1, keepdims=True
        )

        xw = jnp.concatenate(
            [
                xb
                * jnp.where(idx == my * E_LOCAL + e, p_chosen, 0.0).astype(
                    jnp.bfloat16
                )
                for e in range(E_LOCAL)
            ],
            axis=1,
        )
        ew_dma.wait()
        wm = ew_vmem[...].astype(jnp.bfloat16).reshape(E_LOCAL * D_IN, D_OUT)

        def rs_copy(h, d):
            return pltpu.make_async_remote_copy(
                src_ref=acc_ref.at[h * N_DEV + (my + d) % N_DEV],
                dst_ref=rs_buf.at[h * N_PEER + d - 1],
                send_sem=rs_ssem.at[h * N_PEER + d - 1],
                recv_sem=rs_rsem.at[h * N_PEER + d - 1],
                device_id=((my + d) % N_DEV,),
                device_id_type=pl.DeviceIdType.MESH,
            )

        for h in range(N_HALF):
            partial_h = jnp.dot(
                xw, wm[:, h * COLS : (h + 1) * COLS],
                preferred_element_type=jnp.float32,
            )
            acc_ref[h * N_DEV : (h + 1) * N_DEV] = partial_h.reshape(
                N_DEV, ROWS, COLS
            ).astype(jnp.bfloat16)

            if _PHASES == 0:
                continue
            if h == 0:
                pl.semaphore_wait(barrier_sem, N_DEV - 1)
            for d in range(1, N_DEV):
                rs_copy(h, d).start()

        if _PHASES == 0:
            sw_dma.wait()
            out_ref[...] = jnp.concatenate(
                [
                    acc_ref[h * N_DEV : (h + 1) * N_DEV].reshape(N_TOK, COLS)
                    for h in range(N_HALF)
                ],
                axis=1,
            )
            return

        sw_dma.wait()
        shared_own = jnp.dot(
            x_ref[pl.ds(my * ROWS, ROWS), :].astype(jnp.bfloat16),
            sw_vmem[...].astype(jnp.bfloat16),
            preferred_element_type=jnp.float32,
        )

        ag = []
        for h in range(N_HALF):
            rs = [rs_copy(h, d) for d in range(1, N_DEV)]
            for c in rs:
                c.wait_send()
            for c in rs:
                c.wait_recv()

            own = acc_ref[my + h * N_DEV].astype(jnp.float32)
            red = (
                own
                + jnp.sum(
                    rs_buf[h * N_PEER : (h + 1) * N_PEER].astype(jnp.float32),
                    axis=0,
                )
                + shared_own[:, h * COLS : (h + 1) * COLS]
            )
            out_ref[pl.ds(my * ROWS, ROWS), h * COLS : (h + 1) * COLS] = red.astype(
                jnp.bfloat16
            )

            if _PHASES == 1:
                continue
            for d in range(1, N_DEV):
                c = pltpu.make_async_remote_copy(
                    src_ref=out_ref.at[
                        pl.ds(my * ROWS, ROWS), h * COLS : (h + 1) * COLS
                    ],
                    dst_ref=out_ref.at[
                        pl.ds(my * ROWS, ROWS), h * COLS : (h + 1) * COLS
                    ],
                    send_sem=ag_ssem.at[h * N_PEER + d - 1],
                    recv_sem=ag_rsem.at[h * N_PEER + d - 1],
                    device_id=((my + d) % N_DEV,),
                    device_id_type=pl.DeviceIdType.MESH,
                )
                c.start()
                ag.append(c)

        for c in ag:
            c.wait_send()
        for c in ag:
            c.wait_recv()

    return pl.pallas_call(
        body,
        out_shape=jax.ShapeDtypeStruct((N_TOK, D_OUT), jnp.bfloat16),
        in_specs=[pl.BlockSpec(memory_space=pltpu.VMEM)] * 3
        + [pl.BlockSpec(memory_space=pl.ANY)] * 2,
        out_specs=pl.BlockSpec(memory_space=pltpu.VMEM),
        scratch_shapes=[
            pltpu.VMEM((N_HALF * N_DEV, ROWS, COLS), jnp.bfloat16),
            pltpu.VMEM((N_HALF * N_PEER, ROWS, COLS), jnp.bfloat16),
            pltpu.VMEM((E_LOCAL, D_IN, D_OUT), jnp.float32),
            pltpu.VMEM((D_IN, D_OUT), jnp.float32),
            pltpu.SemaphoreType.DMA((2,)),
            pltpu.SemaphoreType.DMA((N_HALF * N_PEER,)),
            pltpu.SemaphoreType.DMA((N_HALF * N_PEER,)),
            pltpu.SemaphoreType.DMA((N_HALF * N_PEER,)),
            pltpu.SemaphoreType.DMA((N_HALF * N_PEER,)),
        ],
        **(
            {"compiler_params": pltpu.CompilerParams(collective_id=0)}
            if _PHASES > 0
            else {}
        ),
    )(x, router_W, route_idx, expert_W, shared_W)
